# Initial kernel scaffold; baseline (speedup 1.0000x reference)
import jax
import jax.numpy as jnp
from jax.experimental import pallas as pl


def kernel(x, edge_index, edge_weight, W1, b1, g1, be1, W2, b2, g2, be2, Wf1, bf1, Wf2, bf2):
    raise NotImplementedError("write your pallas kernel here")


# speedup vs baseline: 1.2282x; 1.2282x over previous
import jax
import jax.numpy as jnp
from jax.experimental import pallas as pl

N = 10000
E = 320000
NFEAT = 128
NHID = 128


def _mm_kernel(a_ref, b_ref, o_ref):
    o_ref[...] = jnp.dot(a_ref[...], b_ref[...], preferred_element_type=jnp.float32)


def _mm(a, b):
    return pl.pallas_call(
        _mm_kernel,
        out_shape=jax.ShapeDtypeStruct((a.shape[0], b.shape[1]), jnp.float32),
    )(a, b)


def _bn_relu_kernel(x_ref, g_ref, b_ref, o_ref):
    h = jnp.maximum(x_ref[...], 0.0)
    m = jnp.mean(h, axis=0, keepdims=True)
    v = jnp.mean((h - m) ** 2, axis=0, keepdims=True)
    o_ref[...] = (h - m) * jax.lax.rsqrt(v + 1e-5) * g_ref[...] + b_ref[...]


def _bn_relu(x, g, b):
    return pl.pallas_call(
        _bn_relu_kernel,
        out_shape=jax.ShapeDtypeStruct(x.shape, jnp.float32),
    )(x, g.reshape(1, -1), b.reshape(1, -1))


def _head_kernel(c_ref, w1_ref, b1_ref, w2_ref, b2_ref, o_ref):
    h = jnp.maximum(jnp.dot(c_ref[...], w1_ref[...], preferred_element_type=jnp.float32) + b1_ref[...], 0.0)
    o_ref[...] = jnp.maximum(jnp.dot(h, w2_ref[...], preferred_element_type=jnp.float32) + b2_ref[...], 0.0)


def kernel(x, edge_index, edge_weight, W1, b1, g1, be1, W2, b2, g2, be2, Wf1, bf1, Wf2, bf2):
    row, col = edge_index[0], edge_index[1]
    w = edge_weight
    deg = jax.ops.segment_sum(w, col, num_segments=N) + 1.0
    dinv = jax.lax.rsqrt(deg)
    norm = dinv[row] * w * dinv[col]
    sl = dinv * dinv

    def conv(h, W, b):
        hw = _mm(h, W)
        agg = jax.ops.segment_sum(norm[:, None] * hw[row], col, num_segments=N)
        return agg + sl[:, None] * hw + b

    h = _bn_relu(conv(x, W1, b1), g1, be1)
    h2 = _bn_relu(conv(h, W2, b2), g2, be2)
    cat = jnp.concatenate([x, h, h2], axis=1)
    o = pl.pallas_call(
        _head_kernel,
        out_shape=jax.ShapeDtypeStruct((N, 1), jnp.float32),
    )(cat, Wf1, bf1.reshape(1, -1), Wf2, bf2.reshape(1, -1))
    return o.reshape(-1)


# speedup vs baseline: 2.7007x; 2.1989x over previous
import jax
import jax.numpy as jnp
from jax import lax
from jax.experimental import pallas as pl
from jax.experimental.pallas import tpu as pltpu
from jax.experimental.pallas import tpu_sc as plsc

N = 10000
E = 320000
F = 128
NC, NS, L = 2, 16, 16
NW = NC * NS

RP = 2560
EP = RP * F
MB = RP // NW
TB = RP // NS
HF = F // NC
NBUF = 2
QB = 32

ZCH = 640
ZLAST = N - (NS - 1) * ZCH
NP = NS * ZCH


def _make_deg():
    mesh = plsc.VectorSubcoreMesh(core_axis_name="c", subcore_axis_name="s")
    out_type = jax.ShapeDtypeStruct((NC, NP), jnp.float32)
    scratch = [
        pltpu.VMEM((MB, F), jnp.int32),
        pltpu.VMEM((MB, F), jnp.float32),
        pltpu.VMEM((ZCH,), jnp.float32),
        pltpu.VMEM_SHARED((NP,), jnp.float32),
    ]

    def body(col2d, w2d, degp, ibuf, fbuf, zbuf, deg):
        cid = lax.axis_index("c")
        sid = lax.axis_index("s")

        zero16 = jnp.zeros((L,), jnp.float32)
        for i in range(ZCH // L):
            zbuf[pl.ds(i * L, L)] = zero16

        pltpu.sync_copy(zbuf, deg.at[pl.ds(sid * ZCH, ZCH)])

        plsc.subcore_barrier()

        d0 = (cid * NS + sid) * MB
        pltpu.sync_copy(col2d.at[pl.ds(d0, MB)], ibuf)
        pltpu.sync_copy(w2d.at[pl.ds(d0, MB)], fbuf)

        def dscat(i, c):
            pltpu.sync_copy(fbuf.at[i], deg.at[ibuf.at[i]], add=True)
            return c
        lax.fori_loop(0, MB, dscat, 0)

        plsc.subcore_barrier()

        pltpu.sync_copy(deg.at[pl.ds(sid * ZCH, ZCH)],
                        degp.at[cid, pl.ds(sid * ZCH, ZCH)])

    return pl.kernel(body, out_type=out_type, mesh=mesh, scratch_types=scratch)


def _make_norm():
    mesh = plsc.VectorSubcoreMesh(core_axis_name="c", subcore_axis_name="s")
    out_type = jax.ShapeDtypeStruct((RP, F), jnp.float32)
    scratch = [
        pltpu.VMEM((2 * MB, F), jnp.int32),
        pltpu.VMEM((MB, F), jnp.float32),
        pltpu.VMEM((2, F), jnp.float32),
        pltpu.VMEM((2, F), jnp.float32),
        pltpu.SemaphoreType.DMA, pltpu.SemaphoreType.DMA,
    ]

    def body(row2d, col2d, w2d, dinv_in, norm2d, ibuf, fbuf, drb, dcb, sr, sc):
        cid = lax.axis_index("c")
        sid = lax.axis_index("s")
        wid = cid * NS + sid

        CO = MB
        m0 = MB * wid
        pltpu.sync_copy(row2d.at[pl.ds(m0, MB)], ibuf.at[pl.ds(0, MB)])
        pltpu.sync_copy(col2d.at[pl.ds(m0, MB)], ibuf.at[pl.ds(CO, MB)])
        pltpu.sync_copy(w2d.at[pl.ds(m0, MB)], fbuf)

        def dfire(i, slot):
            pltpu.async_copy(dinv_in.at[ibuf.at[i]], drb.at[slot], sr)
            pltpu.async_copy(dinv_in.at[ibuf.at[CO + i]], dcb.at[slot], sc)

        def dwait(slot):
            pltpu.make_async_copy(dinv_in.at[ibuf.at[0]], drb.at[slot], sr).wait()
            pltpu.make_async_copy(dinv_in.at[ibuf.at[0]], dcb.at[slot], sc).wait()

        def nrow(i, slot):
            for kk in range(F // L):
                sl = pl.ds(kk * L, L)
                fbuf[i, sl] = drb[slot, sl] * fbuf[i, sl] * dcb[slot, sl]

        dfire(0, 0)

        def nouter(k, c):
            i = 2 * k
            dfire(i + 1, 1)
            dwait(0)
            nrow(i, 0)
            dfire(jnp.minimum(i + 2, MB - 1), 0)
            dwait(1)
            nrow(i + 1, 1)
            return c
        lax.fori_loop(0, MB // 2, nouter, 0)
        dwait(0)

        pltpu.sync_copy(fbuf, norm2d.at[pl.ds(m0, MB)])

    return pl.kernel(body, out_type=out_type, mesh=mesh, scratch_types=scratch)


def _make_edge_agg():
    mesh = plsc.VectorSubcoreMesh(core_axis_name="c", subcore_axis_name="s")
    out_type = jax.ShapeDtypeStruct((NC, N, HF), jnp.float32)

    scratch = [
        pltpu.VMEM((QB, F), jnp.int32),
        pltpu.VMEM((QB, F), jnp.int32),
        pltpu.VMEM((QB, F), jnp.float32),
        pltpu.VMEM((F, F), jnp.float32),
        pltpu.VMEM((F, F), jnp.float32),
        pltpu.VMEM((F, HF), jnp.float32),
        pltpu.VMEM((F, HF), jnp.float32),
        pltpu.VMEM_SHARED((N, HF), jnp.float32),
    ] + [pltpu.SemaphoreType.DMA] * (2 * NBUF)

    def body(xw, row2d, col2d, norm2d, out,
             ribuf, cibuf, fbuf, g0, g1, s0, s1, acc, *sems):
        gbuf = (g0, g1)
        sbuf = (s0, s1)
        gsems = sems[:NBUF]
        ssems = sems[NBUF:]

        cid = lax.axis_index("c")
        sid = lax.axis_index("s")

        zero16 = jnp.zeros((L,), jnp.float32)

        def zrow(i, c):
            for kk in range(HF // L):
                s0[i, pl.ds(kk * L, L)] = zero16
            return c
        lax.fori_loop(0, F, zrow, 0)

        zb = s0
        a0 = sid * ZCH

        @pl.when(sid < NS - 1)
        def _():
            for t in range(ZCH // F):
                pltpu.sync_copy(zb, acc.at[pl.ds(a0 + t * F, F)])

        @pl.when(sid == NS - 1)
        def _():
            for t in range(ZLAST // F):
                pltpu.sync_copy(zb, acc.at[pl.ds(a0 + t * F, F)])
            pltpu.sync_copy(zb.at[pl.ds(0, ZLAST % F)],
                            acc.at[pl.ds(a0 + (ZLAST // F) * F, ZLAST % F)])

        plsc.subcore_barrier()

        m0 = TB * sid

        def gather(i, b):
            pltpu.async_copy(xw.at[ribuf.at[i]], gbuf[b], gsems[b])

        def wait_gather(b):
            pltpu.make_async_copy(xw.at[ribuf.at[0]], gbuf[b], gsems[b]).wait()

        def scale(i, b):
            def sgrp_for(off):
                def sgrp(q, c):
                    nv = fbuf[i, pl.ds(q * L, L)]
                    for jj in range(L):
                        s16 = lax.gather(
                            nv, jnp.full((L, 1), jj, jnp.int32),
                            lax.GatherDimensionNumbers(
                                offset_dims=(), collapsed_slice_dims=(0,),
                                start_index_map=(0,)),
                            (1,), mode=lax.GatherScatterMode.PROMISE_IN_BOUNDS)
                        j = q * L + jj
                        for kk in range(HF // L):
                            sbuf[b][j, pl.ds(kk * L, L)] = (
                                gbuf[b][j, pl.ds(off + kk * L, L)] * s16)
                    return c
                return sgrp

            @pl.when(cid == 0)
            def _():
                lax.fori_loop(0, F // L, sgrp_for(0), 0)

            @pl.when(cid == 1)
            def _():
                lax.fori_loop(0, F // L, sgrp_for(HF), 0)

        def scatter(i, b):
            pltpu.async_copy(sbuf[b], acc.at[cibuf.at[i]], ssems[b], add=True)

        def wait_scatter(b):
            pltpu.make_async_copy(sbuf[b], acc.at[cibuf.at[0]], ssems[b]).wait()

        def batch_body(batch, carry):
            q0 = m0 + batch * QB
            pltpu.sync_copy(row2d.at[pl.ds(q0, QB)], ribuf)
            pltpu.sync_copy(col2d.at[pl.ds(q0, QB)], cibuf)
            pltpu.sync_copy(norm2d.at[pl.ds(q0, QB)], fbuf)

            def step(k, c):
                gather(k, 0)
                wait_gather(0)
                scale(k, 0)
                scatter(k, 0)
                wait_scatter(0)
                return c
            lax.fori_loop(0, QB, step, 0)
            return carry
        lax.fori_loop(0, TB // QB, batch_body, 0)

        plsc.subcore_barrier()

        @pl.when(sid < NS - 1)
        def _():
            pltpu.sync_copy(acc.at[pl.ds(a0, ZCH)], out.at[cid, pl.ds(a0, ZCH)])

        @pl.when(sid == NS - 1)
        def _():
            pltpu.sync_copy(acc.at[pl.ds(a0, ZLAST)], out.at[cid, pl.ds(a0, ZLAST)])

    return pl.kernel(body, out_type=out_type, mesh=mesh, scratch_types=scratch)


_deg_kernel = _make_deg()
_norm_kernel = _make_norm()
_edge_agg = _make_edge_agg()


def _mm_body(a_ref, b_ref, o_ref):
    o_ref[...] = jnp.dot(a_ref[...], b_ref[...], preferred_element_type=jnp.float32)


def _mm(a, b):
    return pl.pallas_call(
        _mm_body,
        out_shape=jax.ShapeDtypeStruct((a.shape[0], b.shape[1]), jnp.float32),
    )(a, b)


def _dinv_body(degp_ref, o_ref):
    o_ref[...] = lax.rsqrt(degp_ref[0] + degp_ref[1] + 1.0)


def _bn(h, g, b):
    m = jnp.mean(h, axis=0, keepdims=True)
    v = jnp.mean((h - m) ** 2, axis=0, keepdims=True)
    return (h - m) * lax.rsqrt(v + 1e-5) * g + b


def _post1_body(parts, xw, dinv, b1, g1, be1, w2, h_ref, hw2_ref):
    d = dinv[...]
    agg = jnp.concatenate([parts[0], parts[1]], axis=1)
    pre = agg + d * d * xw[...] + b1[...]
    h = _bn(jnp.maximum(pre, 0.0), g1[...], be1[...])
    h_ref[...] = h
    hw2_ref[...] = jnp.dot(h, w2[...], preferred_element_type=jnp.float32)


def _post2_body(parts, hw2, dinv, b2, g2, be2, x, h, wf1, bf1, wf2, bf2, o_ref):
    d = dinv[...]
    agg = jnp.concatenate([parts[0], parts[1]], axis=1)
    pre = agg + d * d * hw2[...] + b2[...]
    h2 = _bn(jnp.maximum(pre, 0.0), g2[...], be2[...])
    o1 = (jnp.dot(x[...], wf1[0], preferred_element_type=jnp.float32)
          + jnp.dot(h[...], wf1[1], preferred_element_type=jnp.float32)
          + jnp.dot(h2, wf1[2], preferred_element_type=jnp.float32) + bf1[...])
    o1 = jnp.maximum(o1, 0.0)
    o2 = jnp.dot(o1, wf2[...], preferred_element_type=jnp.float32) + bf2[...]
    o_ref[...] = jnp.maximum(o2, 0.0)


def kernel(x, edge_index, edge_weight, W1, b1, g1, be1, W2, b2, g2, be2, Wf1, bf1, Wf2, bf2):
    pad = EP - E
    row2d = jnp.concatenate(
        [edge_index[0].astype(jnp.int32), jnp.zeros((pad,), jnp.int32)]).reshape(RP, F)
    col2d = jnp.concatenate(
        [edge_index[1].astype(jnp.int32), jnp.zeros((pad,), jnp.int32)]).reshape(RP, F)
    w2d = jnp.concatenate(
        [edge_weight, jnp.zeros((pad,), jnp.float32)]).reshape(RP, F)

    degp = _deg_kernel(col2d, w2d)
    dinv = pl.pallas_call(
        _dinv_body, out_shape=jax.ShapeDtypeStruct((NP,), jnp.float32),
    )(degp)
    dcol = dinv[:N].reshape(N, 1)

    norm2d = _norm_kernel(row2d, col2d, w2d, dinv)
    _row = edge_index[0].astype(jnp.int32)
    _col = edge_index[1].astype(jnp.int32)
    _norm = norm2d.reshape(-1)[:E]

    xw1 = _mm(x, W1)
    _agg1 = jax.ops.segment_sum(_norm[:, None] * xw1[_row], _col, num_segments=N)
    parts1 = jnp.stack([_agg1[:, :HF], _agg1[:, HF:]])

    h, hw2 = pl.pallas_call(
        _post1_body,
        out_shape=[jax.ShapeDtypeStruct((N, F), jnp.float32),
                   jax.ShapeDtypeStruct((N, F), jnp.float32)],
    )(parts1, xw1, dcol, b1.reshape(1, F), g1.reshape(1, F), be1.reshape(1, F), W2)

    _agg2 = jax.ops.segment_sum(_norm[:, None] * hw2[_row], _col, num_segments=N)
    parts2 = jnp.stack([_agg2[:, :HF], _agg2[:, HF:]])

    o = pl.pallas_call(
        _post2_body,
        out_shape=jax.ShapeDtypeStruct((N, 1), jnp.float32),
    )(parts2, hw2, dcol, b2.reshape(1, F), g2.reshape(1, F), be2.reshape(1, F),
      x, h, Wf1.reshape(3, F, F), bf1.reshape(1, F), Wf2, bf2.reshape(1, 1))
    return o.reshape(-1)


# speedup vs baseline: 4.4948x; 1.6643x over previous
import jax
import jax.numpy as jnp
from jax import lax
from jax.experimental import pallas as pl
from jax.experimental.pallas import tpu as pltpu
from jax.experimental.pallas import tpu_sc as plsc

N = 10000
E = 320000
F = 128
NC, NS, L = 2, 16, 16
NW = NC * NS

RP = 2560
EP = RP * F
MB = RP // NW
TB = RP // NS
HF = F // NC
NBUF = 2
QB = 32

ZCH = 640
ZLAST = N - (NS - 1) * ZCH
NP = NS * ZCH


def _make_deg():
    mesh = plsc.VectorSubcoreMesh(core_axis_name="c", subcore_axis_name="s")
    out_type = jax.ShapeDtypeStruct((NC, NP), jnp.float32)
    scratch = [
        pltpu.VMEM((MB, F), jnp.int32),
        pltpu.VMEM((MB, F), jnp.float32),
        pltpu.VMEM((ZCH,), jnp.float32),
        pltpu.VMEM_SHARED((NP,), jnp.float32),
    ]

    def body(col2d, w2d, degp, ibuf, fbuf, zbuf, deg):
        cid = lax.axis_index("c")
        sid = lax.axis_index("s")

        zero16 = jnp.zeros((L,), jnp.float32)
        for i in range(ZCH // L):
            zbuf[pl.ds(i * L, L)] = zero16

        pltpu.sync_copy(zbuf, deg.at[pl.ds(sid * ZCH, ZCH)])

        plsc.subcore_barrier()

        d0 = (cid * NS + sid) * MB
        pltpu.sync_copy(col2d.at[pl.ds(d0, MB)], ibuf)
        pltpu.sync_copy(w2d.at[pl.ds(d0, MB)], fbuf)

        def dscat(i, c):
            pltpu.sync_copy(fbuf.at[i], deg.at[ibuf.at[i]], add=True)
            return c
        lax.fori_loop(0, MB, dscat, 0)

        plsc.subcore_barrier()

        pltpu.sync_copy(deg.at[pl.ds(sid * ZCH, ZCH)],
                        degp.at[cid, pl.ds(sid * ZCH, ZCH)])

    return pl.kernel(body, out_type=out_type, mesh=mesh, scratch_types=scratch)


def _make_norm():
    mesh = plsc.VectorSubcoreMesh(core_axis_name="c", subcore_axis_name="s")
    out_type = jax.ShapeDtypeStruct((RP, F), jnp.float32)
    scratch = [
        pltpu.VMEM((2 * MB, F), jnp.int32),
        pltpu.VMEM((MB, F), jnp.float32),
        pltpu.VMEM((2, F), jnp.float32),
        pltpu.VMEM((2, F), jnp.float32),
        pltpu.SemaphoreType.DMA, pltpu.SemaphoreType.DMA,
    ]

    def body(row2d, col2d, w2d, dinv_in, norm2d, ibuf, fbuf, drb, dcb, sr, sc):
        cid = lax.axis_index("c")
        sid = lax.axis_index("s")
        wid = cid * NS + sid

        CO = MB
        m0 = MB * wid
        pltpu.sync_copy(row2d.at[pl.ds(m0, MB)], ibuf.at[pl.ds(0, MB)])
        pltpu.sync_copy(col2d.at[pl.ds(m0, MB)], ibuf.at[pl.ds(CO, MB)])
        pltpu.sync_copy(w2d.at[pl.ds(m0, MB)], fbuf)

        def dfire(i, slot):
            pltpu.async_copy(dinv_in.at[ibuf.at[i]], drb.at[slot], sr)
            pltpu.async_copy(dinv_in.at[ibuf.at[CO + i]], dcb.at[slot], sc)

        def dwait(slot):
            pltpu.make_async_copy(dinv_in.at[ibuf.at[0]], drb.at[slot], sr).wait()
            pltpu.make_async_copy(dinv_in.at[ibuf.at[0]], dcb.at[slot], sc).wait()

        def nrow(i, slot):
            for kk in range(F // L):
                sl = pl.ds(kk * L, L)
                fbuf[i, sl] = drb[slot, sl] * fbuf[i, sl] * dcb[slot, sl]

        dfire(0, 0)

        def nouter(k, c):
            i = 2 * k
            dfire(i + 1, 1)
            dwait(0)
            nrow(i, 0)
            dfire(jnp.minimum(i + 2, MB - 1), 0)
            dwait(1)
            nrow(i + 1, 1)
            return c
        lax.fori_loop(0, MB // 2, nouter, 0)
        dwait(0)

        pltpu.sync_copy(fbuf, norm2d.at[pl.ds(m0, MB)])

    return pl.kernel(body, out_type=out_type, mesh=mesh, scratch_types=scratch)


NH = N // NC
AR = 5128
AZ = AR // NS


def _make_edge_agg():
    mesh = plsc.VectorSubcoreMesh(core_axis_name="c", subcore_axis_name="s")
    out_type = jax.ShapeDtypeStruct((NC, AR, F), jnp.float32)

    scratch = [
        pltpu.VMEM((QB, F), jnp.int32),
        pltpu.VMEM((QB, F), jnp.int32),
        pltpu.VMEM((QB, F), jnp.float32),
        pltpu.VMEM((F, F), jnp.float32),
        pltpu.VMEM_SHARED((AR, F), jnp.float32),
    ] + [pltpu.SemaphoreType.DMA] * 2

    def body(xw, row2d, col2d, norm2d, out, ribuf, cibuf, fbuf, gbuf, acc, gsem, ssem):
        cid = lax.axis_index("c")
        sid = lax.axis_index("s")
        base = cid * NH

        zero16 = jnp.zeros((L,), jnp.float32)

        def zrow(i, c):
            for kk in range(F // L):
                gbuf[i, pl.ds(kk * L, L)] = zero16
            return c
        lax.fori_loop(0, F, zrow, 0)

        a0 = sid * 320
        for t in range(320 // F):
            pltpu.sync_copy(gbuf, acc.at[pl.ds(a0 + t * F, F)])
        pltpu.sync_copy(gbuf.at[pl.ds(0, 320 % F)],
                        acc.at[pl.ds(a0 + (320 // F) * F, 320 % F)])

        @pl.when(sid == 0)
        def _():
            pltpu.sync_copy(gbuf.at[pl.ds(0, AR - NS * 320)],
                            acc.at[pl.ds(NS * 320, AR - NS * 320)])

        plsc.subcore_barrier()

        m0 = TB * sid

        def batch_body(batch, carry):
            q0 = m0 + batch * QB
            pltpu.sync_copy(row2d.at[pl.ds(q0, QB)], ribuf)
            pltpu.sync_copy(col2d.at[pl.ds(q0, QB)], cibuf)
            pltpu.sync_copy(norm2d.at[pl.ds(q0, QB)], fbuf)

            def rebase(i, c):
                for kk in range(F // L):
                    sl = pl.ds(kk * L, L)
                    t = cibuf[i, sl] - base
                    oob = (t < 0) | (t >= NH)
                    cibuf[i, sl] = jnp.where(oob, NH, t)
                return c
            lax.fori_loop(0, QB, rebase, 0)

            def scale(i):
                def sgrp(q, c):
                    nv = fbuf[i, pl.ds(q * L, L)]
                    for jj in range(L):
                        s16 = lax.gather(
                            nv, jnp.full((L, 1), jj, jnp.int32),
                            lax.GatherDimensionNumbers(
                                offset_dims=(), collapsed_slice_dims=(0,),
                                start_index_map=(0,)),
                            (1,), mode=lax.GatherScatterMode.PROMISE_IN_BOUNDS)
                        j = q * L + jj
                        for kk in range(F // L):
                            sl = pl.ds(kk * L, L)
                            gbuf[j, sl] = gbuf[j, sl] * s16
                    return c
                lax.fori_loop(0, F // L, sgrp, 0)

            def step(k, c):
                pltpu.async_copy(xw.at[ribuf.at[k]], gbuf, gsem)
                pltpu.make_async_copy(xw.at[ribuf.at[0]], gbuf, gsem).wait()
                scale(k)
                pltpu.async_copy(gbuf, acc.at[cibuf.at[k]], ssem, add=True)
                pltpu.make_async_copy(gbuf, acc.at[cibuf.at[0]], ssem).wait()
                return c
            lax.fori_loop(0, QB, step, 0)
            return carry
        lax.fori_loop(0, TB // QB, batch_body, 0)

        plsc.subcore_barrier()

        pltpu.sync_copy(acc.at[pl.ds(a0, 320)], out.at[cid, pl.ds(a0, 320)])

        @pl.when(sid == 0)
        def _():
            pltpu.sync_copy(acc.at[pl.ds(NS * 320, AR - NS * 320)],
                            out.at[cid, pl.ds(NS * 320, AR - NS * 320)])

    return pl.kernel(body, out_type=out_type, mesh=mesh, scratch_types=scratch)


_deg_kernel = _make_deg()
_norm_kernel = _make_norm()
_edge_agg = _make_edge_agg()


def _mm_body(a_ref, b_ref, o_ref):
    o_ref[...] = jnp.dot(a_ref[...], b_ref[...], preferred_element_type=jnp.float32)


def _mm(a, b):
    return pl.pallas_call(
        _mm_body,
        out_shape=jax.ShapeDtypeStruct((a.shape[0], b.shape[1]), jnp.float32),
    )(a, b)


def _dinv_body(degp_ref, o_ref):
    o_ref[...] = lax.rsqrt(degp_ref[0] + degp_ref[1] + 1.0)


def _bn(h, g, b):
    m = jnp.mean(h, axis=0, keepdims=True)
    v = jnp.mean((h - m) ** 2, axis=0, keepdims=True)
    return (h - m) * lax.rsqrt(v + 1e-5) * g + b


def _post1_body(parts, xw, dinv, b1, g1, be1, w2, h_ref, hw2_ref):
    d = dinv[...]
    agg = jnp.concatenate([parts[0, :NH], parts[1, :NH]], axis=0)
    pre = agg + d * d * xw[...] + b1[...]
    h = _bn(jnp.maximum(pre, 0.0), g1[...], be1[...])
    h_ref[...] = h
    hw2_ref[...] = jnp.dot(h, w2[...], preferred_element_type=jnp.float32)


def _post2_body(parts, hw2, dinv, b2, g2, be2, x, h, wf1, bf1, wf2, bf2, o_ref):
    d = dinv[...]
    agg = jnp.concatenate([parts[0, :NH], parts[1, :NH]], axis=0)
    pre = agg + d * d * hw2[...] + b2[...]
    h2 = _bn(jnp.maximum(pre, 0.0), g2[...], be2[...])
    o1 = (jnp.dot(x[...], wf1[0], preferred_element_type=jnp.float32)
          + jnp.dot(h[...], wf1[1], preferred_element_type=jnp.float32)
          + jnp.dot(h2, wf1[2], preferred_element_type=jnp.float32) + bf1[...])
    o1 = jnp.maximum(o1, 0.0)
    o2 = jnp.dot(o1, wf2[...], preferred_element_type=jnp.float32) + bf2[...]
    o_ref[...] = jnp.maximum(o2, 0.0)


def kernel(x, edge_index, edge_weight, W1, b1, g1, be1, W2, b2, g2, be2, Wf1, bf1, Wf2, bf2):
    pad = EP - E
    row2d = jnp.concatenate(
        [edge_index[0].astype(jnp.int32), jnp.zeros((pad,), jnp.int32)]).reshape(RP, F)
    col2d = jnp.concatenate(
        [edge_index[1].astype(jnp.int32), jnp.zeros((pad,), jnp.int32)]).reshape(RP, F)
    w2d = jnp.concatenate(
        [edge_weight, jnp.zeros((pad,), jnp.float32)]).reshape(RP, F)

    degp = _deg_kernel(col2d, w2d)
    dinv = pl.pallas_call(
        _dinv_body, out_shape=jax.ShapeDtypeStruct((NP,), jnp.float32),
    )(degp)
    dcol = dinv[:N].reshape(N, 1)

    norm2d = _norm_kernel(row2d, col2d, w2d, dinv)

    xw1 = _mm(x, W1)
    parts1 = _edge_agg(xw1, row2d, col2d, norm2d)

    h, hw2 = pl.pallas_call(
        _post1_body,
        out_shape=[jax.ShapeDtypeStruct((N, F), jnp.float32),
                   jax.ShapeDtypeStruct((N, F), jnp.float32)],
    )(parts1, xw1, dcol, b1.reshape(1, F), g1.reshape(1, F), be1.reshape(1, F), W2)

    parts2 = _edge_agg(hw2, row2d, col2d, norm2d)

    o = pl.pallas_call(
        _post2_body,
        out_shape=jax.ShapeDtypeStruct((N, 1), jnp.float32),
    )(parts2, hw2, dcol, b2.reshape(1, F), g2.reshape(1, F), be2.reshape(1, F),
      x, h, Wf1.reshape(3, F, F), bf1.reshape(1, F), Wf2, bf2.reshape(1, 1))
    return o.reshape(-1)
